# Initial kernel scaffold; baseline (speedup 1.0000x reference)
import jax
import jax.numpy as jnp
from jax.experimental import pallas as pl


def kernel(wft_ics, wft_vals, bft_ics, bft_vals, stm, ft_w, ft_b, fc1_w, fc1_b, fc2_w, fc2_b, fco_w, fco_b):
    raise NotImplementedError("write your pallas kernel here")


# speedup vs baseline: 7.8729x; 7.8729x over previous
import functools

import jax
import jax.numpy as jnp
from jax import lax
from jax.experimental import pallas as pl
from jax.experimental.pallas import tpu as pltpu
from jax.experimental.pallas import tpu_sc as plsc

_B = 4096
_F = 32
_DM = 256
_DP = 272
_L = 16


def _ft_sc_call(wics, wvals, bics, bvals, ftw_padded):
    info = plsc.get_sparse_core_info()
    nw = info.num_cores * info.num_subcores
    bpw = _B // nw
    nchunk = _DP // _L
    mesh = plsc.VectorSubcoreMesh(core_axis_name="c", subcore_axis_name="s")

    @functools.partial(
        pl.kernel,
        out_type=(
            jax.ShapeDtypeStruct((_B, _DP), jnp.float32),
            jax.ShapeDtypeStruct((_B, _DP), jnp.float32),
        ),
        mesh=mesh,
        compiler_params=pltpu.CompilerParams(use_tc_tiling_on_sc=False,
                                             needs_layout_passes=False),
        scratch_types=[
            pltpu.VMEM((bpw, _F), jnp.int32),
            pltpu.VMEM((bpw, _F), jnp.int32),
            pltpu.VMEM((bpw, _F), jnp.float32),
            pltpu.VMEM((bpw, _F), jnp.float32),
            pltpu.VMEM((_F, _DP), jnp.float32),
            pltpu.VMEM((bpw, _DP), jnp.float32),
            pltpu.VMEM((bpw, _DP), jnp.float32),
            pltpu.SemaphoreType.DMA,
        ],
    )
    def ft_kernel(wics_h, wvals_h, bics_h, bvals_h, ftw_h, wf_o, bf_o,
                  widx_v, bidx_v, wval_v, bval_v, rows_v, wacc_v, bacc_v, sem):
        wid = lax.axis_index("s") * info.num_cores + lax.axis_index("c")
        base = wid * bpw
        pltpu.sync_copy(wics_h.at[pl.ds(base, bpw)], widx_v)
        pltpu.sync_copy(bics_h.at[pl.ds(base, bpw)], bidx_v)
        pltpu.sync_copy(wvals_h.at[pl.ds(base, bpw)], wval_v)
        pltpu.sync_copy(bvals_h.at[pl.ds(base, bpw)], bval_v)

        def side_body(idx_v, val_v, acc_v):
            def unit(i, carry):
                pltpu.async_copy(ftw_h.at[idx_v.at[i]], rows_v, sem).wait()
                v0 = val_v[i, pl.ds(0, _L)]
                v1 = val_v[i, pl.ds(_L, _L)]
                accs = [jnp.zeros((_L,), jnp.float32) for _ in range(nchunk)]
                for r in range(_F):
                    v = v0[r] if r < _L else v1[r - _L]
                    for c in range(nchunk):
                        accs[c] = accs[c] + rows_v[r, pl.ds(c * _L, _L)] * v
                for c in range(nchunk):
                    acc_v[i, pl.ds(c * _L, _L)] = accs[c]
                return carry
            lax.fori_loop(0, bpw, unit, 0)

        side_body(widx_v, wval_v, wacc_v)
        side_body(bidx_v, bval_v, bacc_v)

        pltpu.sync_copy(wacc_v, wf_o.at[pl.ds(base, bpw)])
        pltpu.sync_copy(bacc_v, bf_o.at[pl.ds(base, bpw)])

    return ft_kernel(wics, wvals, bics, bvals, ftw_padded)


def _head_body(wf_r, bf_r, stm_r, ftb_r, w1a_r, w1b_r, b1_r, w2_r,
               b2_r, wo_r, bo_r, o_r):
    stm = stm_r[...]
    wfull = wf_r[...]
    bfull = bf_r[...]
    wfv = wfull[:, :_DM] + ftb_r[...]
    bfv = bfull[:, :_DM] + ftb_r[...]
    pd = wfull[:, _DM:_DM + 1] - bfull[:, _DM:_DM + 1]
    x1 = jnp.clip(wfv + stm * (bfv - wfv), 0.0, 1.0)
    x2 = jnp.clip(bfv + stm * (wfv - bfv), 0.0, 1.0)
    h = jnp.dot(x1, w1a_r[...], preferred_element_type=jnp.float32)
    h = h + jnp.dot(x2, w1b_r[...], preferred_element_type=jnp.float32)
    h = jnp.clip(h + b1_r[...], 0.0, 1.0)
    h = jnp.clip(jnp.dot(h, w2_r[...], preferred_element_type=jnp.float32) + b2_r[...], 0.0, 1.0)
    y = jnp.dot(h, wo_r[...], preferred_element_type=jnp.float32) + bo_r[...]
    o_r[...] = y + pd * (0.5 - stm)


def _head_tc_call(wf, bf, stm, ftb, w1a, w1b, b1, w2, b2, wo, bo):
    bk = 512
    grid = (_B // bk,)
    row_spec = lambda w: pl.BlockSpec((bk, w), lambda i: (i, 0))
    full_spec = lambda a: pl.BlockSpec(a.shape, lambda i: tuple(0 for _ in a.shape))
    return pl.pallas_call(
        _head_body,
        grid=grid,
        in_specs=[
            row_spec(_DP), row_spec(_DP), row_spec(1),
            full_spec(ftb), full_spec(w1a), full_spec(w1b), full_spec(b1),
            full_spec(w2), full_spec(b2), full_spec(wo), full_spec(bo),
        ],
        out_specs=row_spec(1),
        out_shape=jax.ShapeDtypeStruct((_B, 1), jnp.float32),
    )(wf, bf, stm, ftb, w1a, w1b, b1, w2, b2, wo, bo)


def kernel(wft_ics, wft_vals, bft_ics, bft_vals, stm, ft_w, ft_b,
           fc1_w, fc1_b, fc2_w, fc2_b, fco_w, fco_b):
    ftw_padded = jnp.pad(ft_w, ((0, 0), (0, _DP - ft_w.shape[1])))
    wf, bf = _ft_sc_call(wft_ics, wft_vals, bft_ics, bft_vals, ftw_padded)
    ftb = ft_b[:_DM].reshape(1, _DM)
    w1a = fc1_w[:, :_DM].T
    w1b = fc1_w[:, _DM:].T
    return _head_tc_call(
        wf, bf, stm, ftb, w1a, w1b,
        fc1_b.reshape(1, 32), fc2_w.T, fc2_b.reshape(1, 32),
        fco_w.T, fco_b.reshape(1, 1))


# speedup vs baseline: 8.9985x; 1.1430x over previous
import functools

import jax
import jax.numpy as jnp
from jax import lax
from jax.experimental import pallas as pl
from jax.experimental.pallas import tpu as pltpu
from jax.experimental.pallas import tpu_sc as plsc

_B = 4096
_F = 32
_DM = 256
_DP = 272
_L = 16
_NBUF = 2


def _ft_sc_call(wics, wvals, bics, bvals, ftw_padded):
    info = plsc.get_sparse_core_info()
    nw = info.num_cores * info.num_subcores
    bpw = _B // nw
    nunit = 2 * bpw
    nchunk = _DP // _L
    mesh = plsc.VectorSubcoreMesh(core_axis_name="c", subcore_axis_name="s")

    @functools.partial(
        pl.kernel,
        out_type=(
            jax.ShapeDtypeStruct((_B, _DP), jnp.float32),
            jax.ShapeDtypeStruct((_B, _DP), jnp.float32),
        ),
        mesh=mesh,
        compiler_params=pltpu.CompilerParams(use_tc_tiling_on_sc=False,
                                             needs_layout_passes=False),
        scratch_types=[
            pltpu.VMEM((nunit, _F), jnp.int32),
            pltpu.VMEM((nunit, _F), jnp.float32),
            [pltpu.VMEM((_F, _DP), jnp.float32)] * _NBUF,
            pltpu.VMEM((nunit, _DP), jnp.float32),
            [pltpu.SemaphoreType.DMA] * _NBUF,
        ],
    )
    def ft_kernel(wics_h, wvals_h, bics_h, bvals_h, ftw_h, wf_o, bf_o,
                  cidx_v, cval_v, rows, acc_v, sems):
        wid = lax.axis_index("s") * info.num_cores + lax.axis_index("c")
        base = wid * bpw
        pltpu.sync_copy(wics_h.at[pl.ds(base, bpw)], cidx_v.at[pl.ds(0, bpw)])
        pltpu.sync_copy(bics_h.at[pl.ds(base, bpw)], cidx_v.at[pl.ds(bpw, bpw)])
        pltpu.sync_copy(wvals_h.at[pl.ds(base, bpw)], cval_v.at[pl.ds(0, bpw)])
        pltpu.sync_copy(bvals_h.at[pl.ds(base, bpw)], cval_v.at[pl.ds(bpw, bpw)])

        def fire(u, b):
            pltpu.async_copy(ftw_h.at[cidx_v.at[u]], rows[b], sems[b])

        def consume(u, b):
            pltpu.make_async_copy(ftw_h.at[cidx_v.at[u]], rows[b], sems[b]).wait()
            v0 = cval_v[u, pl.ds(0, _L)]
            v1 = cval_v[u, pl.ds(_L, _L)]
            accs = [jnp.zeros((_L,), jnp.float32) for _ in range(nchunk)]
            for r in range(_F):
                v = v0[r] if r < _L else v1[r - _L]
                for c in range(nchunk):
                    accs[c] = accs[c] + rows[b][r, pl.ds(c * _L, _L)] * v
            for c in range(nchunk):
                acc_v[u, pl.ds(c * _L, _L)] = accs[c]

        for b in range(_NBUF - 1):
            fire(b, b)

        def group(k, carry):
            u0 = k * _NBUF
            for b in range(_NBUF):
                u = u0 + b
                nxt = u + _NBUF - 1
                bn = (b + _NBUF - 1) % _NBUF

                @pl.when(nxt < nunit)
                def _():
                    fire(nxt, bn)

                consume(u, b)
            return carry

        lax.fori_loop(0, nunit // _NBUF, group, 0)

        pltpu.sync_copy(acc_v.at[pl.ds(0, bpw)], wf_o.at[pl.ds(base, bpw)])
        pltpu.sync_copy(acc_v.at[pl.ds(bpw, bpw)], bf_o.at[pl.ds(base, bpw)])

    return ft_kernel(wics, wvals, bics, bvals, ftw_padded)


def _head_body(wf_r, bf_r, stm_r, ftb_r, w1a_r, w1b_r, b1_r, w2_r,
               b2_r, wo_r, bo_r, o_r):
    stm = stm_r[...]
    wfull = wf_r[...]
    bfull = bf_r[...]
    wfv = wfull[:, :_DM] + ftb_r[...]
    bfv = bfull[:, :_DM] + ftb_r[...]
    pd = wfull[:, _DM:_DM + 1] - bfull[:, _DM:_DM + 1]
    x1 = jnp.clip(wfv + stm * (bfv - wfv), 0.0, 1.0)
    x2 = jnp.clip(bfv + stm * (wfv - bfv), 0.0, 1.0)
    h = jnp.dot(x1, w1a_r[...], preferred_element_type=jnp.float32)
    h = h + jnp.dot(x2, w1b_r[...], preferred_element_type=jnp.float32)
    h = jnp.clip(h + b1_r[...], 0.0, 1.0)
    h = jnp.clip(jnp.dot(h, w2_r[...], preferred_element_type=jnp.float32) + b2_r[...], 0.0, 1.0)
    y = jnp.dot(h, wo_r[...], preferred_element_type=jnp.float32) + bo_r[...]
    o_r[...] = y + pd * (0.5 - stm)


def _head_tc_call(wf, bf, stm, ftb, w1a, w1b, b1, w2, b2, wo, bo):
    bk = 512
    grid = (_B // bk,)
    row_spec = lambda w: pl.BlockSpec((bk, w), lambda i: (i, 0))
    full_spec = lambda a: pl.BlockSpec(a.shape, lambda i: tuple(0 for _ in a.shape))
    return pl.pallas_call(
        _head_body,
        grid=grid,
        in_specs=[
            row_spec(_DP), row_spec(_DP), row_spec(1),
            full_spec(ftb), full_spec(w1a), full_spec(w1b), full_spec(b1),
            full_spec(w2), full_spec(b2), full_spec(wo), full_spec(bo),
        ],
        out_specs=row_spec(1),
        out_shape=jax.ShapeDtypeStruct((_B, 1), jnp.float32),
    )(wf, bf, stm, ftb, w1a, w1b, b1, w2, b2, wo, bo)


def kernel(wft_ics, wft_vals, bft_ics, bft_vals, stm, ft_w, ft_b,
           fc1_w, fc1_b, fc2_w, fc2_b, fco_w, fco_b):
    ftw_padded = jnp.pad(ft_w, ((0, 0), (0, _DP - ft_w.shape[1])))
    wf, bf = _ft_sc_call(wft_ics, wft_vals, bft_ics, bft_vals, ftw_padded)
    ftb = ft_b[:_DM].reshape(1, _DM)
    w1a = fc1_w[:, :_DM].T
    w1b = fc1_w[:, _DM:].T
    return _head_tc_call(
        wf, bf, stm, ftb, w1a, w1b,
        fc1_b.reshape(1, 32), fc2_w.T, fc2_b.reshape(1, 32),
        fco_w.T, fco_b.reshape(1, 1))


# speedup vs baseline: 15.0379x; 1.6712x over previous
import functools

import jax
import jax.numpy as jnp
from jax import lax
from jax.experimental import pallas as pl
from jax.experimental.pallas import tpu as pltpu
from jax.experimental.pallas import tpu_sc as plsc

_B = 4096
_F = 32
_DM = 256
_DP = 272
_L = 16
_NBUF = 4
_RUNROLL = 4


def _ft_sc_call(wics, wvals, bics, bvals, ftw_padded):
    info = plsc.get_sparse_core_info()
    nw = info.num_cores * info.num_subcores
    bpw = _B // nw
    nunit = 2 * bpw
    nchunk = _DP // _L
    mesh = plsc.VectorSubcoreMesh(core_axis_name="c", subcore_axis_name="s")

    @functools.partial(
        pl.kernel,
        out_type=(
            jax.ShapeDtypeStruct((_B, _DP), jnp.float32),
            jax.ShapeDtypeStruct((_B, _DP), jnp.float32),
        ),
        mesh=mesh,
        compiler_params=pltpu.CompilerParams(use_tc_tiling_on_sc=False,
                                             needs_layout_passes=False),
        scratch_types=[
            pltpu.VMEM((nunit, _F), jnp.int32),
            pltpu.VMEM((nunit, _F), jnp.float32),
            [pltpu.VMEM((_F, _DP), jnp.float32)] * _NBUF,
            pltpu.VMEM((nunit, _DP), jnp.float32),
            [pltpu.SemaphoreType.DMA] * _NBUF,
        ],
    )
    def ft_kernel(wics_h, wvals_h, bics_h, bvals_h, ftw_h, wf_o, bf_o,
                  cidx_v, cval_v, rows, acc_v, sems):
        wid = lax.axis_index("s") * info.num_cores + lax.axis_index("c")
        base = wid * bpw
        pltpu.sync_copy(wics_h.at[pl.ds(base, bpw)], cidx_v.at[pl.ds(0, bpw)])
        pltpu.sync_copy(bics_h.at[pl.ds(base, bpw)], cidx_v.at[pl.ds(bpw, bpw)])
        pltpu.sync_copy(wvals_h.at[pl.ds(base, bpw)], cval_v.at[pl.ds(0, bpw)])
        pltpu.sync_copy(bvals_h.at[pl.ds(base, bpw)], cval_v.at[pl.ds(bpw, bpw)])

        def fire(u, b):
            pltpu.async_copy(ftw_h.at[cidx_v.at[u]], rows[b], sems[b])

        def consume(u, b):
            pltpu.make_async_copy(ftw_h.at[cidx_v.at[u]], rows[b], sems[b]).wait()
            uvec = jnp.full((_L,), u, jnp.int32)

            def rstep(k, accs):
                r0 = k * _RUNROLL
                accs = list(accs)
                for dr in range(_RUNROLL):
                    r = r0 + dr
                    vv = plsc.load_gather(cval_v, [uvec, jnp.full((_L,), r, jnp.int32)])
                    for c in range(nchunk):
                        accs[c] = accs[c] + rows[b][r, pl.ds(c * _L, _L)] * vv
                return tuple(accs)

            init = tuple(jnp.zeros((_L,), jnp.float32) for _ in range(nchunk))
            accs = lax.fori_loop(0, _F // _RUNROLL, rstep, init)
            for c in range(nchunk):
                acc_v[u, pl.ds(c * _L, _L)] = accs[c]

        for b in range(_NBUF - 1):
            fire(b, b)

        def group(k, carry):
            u0 = k * _NBUF
            for b in range(_NBUF):
                u = u0 + b
                nxt = u + _NBUF - 1
                bn = (b + _NBUF - 1) % _NBUF

                @pl.when(nxt < nunit)
                def _():
                    fire(nxt, bn)

                consume(u, b)
            return carry

        lax.fori_loop(0, nunit // _NBUF, group, 0)

        pltpu.sync_copy(acc_v.at[pl.ds(0, bpw)], wf_o.at[pl.ds(base, bpw)])
        pltpu.sync_copy(acc_v.at[pl.ds(bpw, bpw)], bf_o.at[pl.ds(base, bpw)])

    return ft_kernel(wics, wvals, bics, bvals, ftw_padded)


def _head_body(wf_r, bf_r, stm_r, ftb_r, w1a_r, w1b_r, b1_r, w2_r,
               b2_r, wo_r, bo_r, o_r):
    stm = stm_r[...]
    wfull = wf_r[...]
    bfull = bf_r[...]
    wfv = wfull[:, :_DM] + ftb_r[...]
    bfv = bfull[:, :_DM] + ftb_r[...]
    pd = wfull[:, _DM:_DM + 1] - bfull[:, _DM:_DM + 1]
    x1 = jnp.clip(wfv + stm * (bfv - wfv), 0.0, 1.0)
    x2 = jnp.clip(bfv + stm * (wfv - bfv), 0.0, 1.0)
    h = jnp.dot(x1, w1a_r[...], preferred_element_type=jnp.float32)
    h = h + jnp.dot(x2, w1b_r[...], preferred_element_type=jnp.float32)
    h = jnp.clip(h + b1_r[...], 0.0, 1.0)
    h = jnp.clip(jnp.dot(h, w2_r[...], preferred_element_type=jnp.float32) + b2_r[...], 0.0, 1.0)
    y = jnp.dot(h, wo_r[...], preferred_element_type=jnp.float32) + bo_r[...]
    o_r[...] = y + pd * (0.5 - stm)


def _head_tc_call(wf, bf, stm, ftb, w1a, w1b, b1, w2, b2, wo, bo):
    bk = 512
    grid = (_B // bk,)
    row_spec = lambda w: pl.BlockSpec((bk, w), lambda i: (i, 0))
    full_spec = lambda a: pl.BlockSpec(a.shape, lambda i: tuple(0 for _ in a.shape))
    return pl.pallas_call(
        _head_body,
        grid=grid,
        in_specs=[
            row_spec(_DP), row_spec(_DP), row_spec(1),
            full_spec(ftb), full_spec(w1a), full_spec(w1b), full_spec(b1),
            full_spec(w2), full_spec(b2), full_spec(wo), full_spec(bo),
        ],
        out_specs=row_spec(1),
        out_shape=jax.ShapeDtypeStruct((_B, 1), jnp.float32),
    )(wf, bf, stm, ftb, w1a, w1b, b1, w2, b2, wo, bo)


def kernel(wft_ics, wft_vals, bft_ics, bft_vals, stm, ft_w, ft_b,
           fc1_w, fc1_b, fc2_w, fc2_b, fco_w, fco_b):
    ftw_padded = jnp.pad(ft_w, ((0, 0), (0, _DP - ft_w.shape[1])))
    wf, bf = _ft_sc_call(wft_ics, wft_vals, bft_ics, bft_vals, ftw_padded)
    ftb = ft_b[:_DM].reshape(1, _DM)
    w1a = fc1_w[:, :_DM].T
    w1b = fc1_w[:, _DM:].T
    return _head_tc_call(
        wf, bf, stm, ftb, w1a, w1b,
        fc1_b.reshape(1, 32), fc2_w.T, fc2_b.reshape(1, 32),
        fco_w.T, fco_b.reshape(1, 1))
